# Initial kernel scaffold; baseline (speedup 1.0000x reference)
import jax
import jax.numpy as jnp
from jax.experimental import pallas as pl


def kernel(pillar_features, voxel_coords):
    raise NotImplementedError("write your pallas kernel here")


# speedup vs baseline: 3.8289x; 3.8289x over previous
import functools

import jax
import jax.numpy as jnp
from jax import lax
from jax.experimental import pallas as pl
from jax.experimental.pallas import tpu as pltpu
from jax.experimental.pallas import tpu_sc as plsc

NX, NY, NZ = 432, 496, 1
C = 64
BATCH = 4
PPS = 16000
P = BATCH * PPS
S = NY * NX
NC, NS = 2, 16
NW = NC * NS
NPB = NW // BATCH
RNG = S // NPB
CK = 2000
NCHUNK = PPS // CK
ZSLOT = PPS
FB = PPS + 16

_mesh = plsc.VectorSubcoreMesh(
    core_axis_name="core", subcore_axis_name="sub",
    num_cores=NC, num_subcores=NS)


@functools.partial(
    pl.kernel,
    mesh=_mesh,
    out_type=jax.ShapeDtypeStruct((BATCH * C * S,), jnp.float32),
    compiler_params=pltpu.CompilerParams(needs_layout_passes=False),
    scratch_types=[
        pltpu.VMEM((RNG,), jnp.int32),
        pltpu.VMEM((CK * 4,), jnp.int32),
        pltpu.VMEM((FB,), jnp.float32),
        pltpu.VMEM((RNG,), jnp.float32),
        pltpu.VMEM((16,), jnp.int32),
    ],
)
def _pp_scatter(feat_t, coords, out, w_v, coords_v, feat_v, out_v, bounce_v):
    wid = lax.axis_index("sub") * NC + lax.axis_index("core")
    b = wid // NPB
    lo = (wid % NPB) * RNG
    lanes = lax.iota(jnp.int32, 16)

    def init_body(i, carry):
        w_v[pl.ds(i * 16, 16)] = jnp.full((16,), ZSLOT, jnp.int32)
        return carry
    lax.fori_loop(0, RNG // 16, init_body, 0)

    def chunk_body(k, carry):
        pltpu.sync_copy(coords.at[pl.ds((b * PPS + k * CK) * 4, CK * 4)],
                        coords_v)

        def vec_body(j, inner):
            row = j * 16 + lanes
            y = plsc.load_gather(coords_v, [row * 4 + 2])
            x = plsc.load_gather(coords_v, [row * 4 + 3])
            s = y * NX + x
            q = k * CK + row
            key = plsc.bitcast(jnp.left_shift(s, 14) | q, jnp.uint32)
            skey, sq = plsc.sort_key_val(key, q)
            ss = plsc.bitcast(jnp.right_shift(skey, jnp.uint32(14)), jnp.int32)
            bounce_v[...] = ss
            nxt = plsc.load_gather(bounce_v, [jnp.minimum(lanes + 1, 15)])
            win = (ss != nxt) | (lanes == 15)
            mask = win & (ss >= lo) & (ss < lo + RNG)
            idxs = jnp.clip(ss - lo, 0, RNG - 1)
            plsc.store_scatter(w_v, [idxs], sq, mask=mask)
            return inner
        lax.fori_loop(0, CK // 16, vec_body, 0)
        return carry
    lax.fori_loop(0, NCHUNK, chunk_body, 0)

    feat_v[pl.ds(ZSLOT, 16)] = jnp.zeros((16,), jnp.float32)

    def chan_body(c, carry):
        pltpu.sync_copy(feat_t.at[pl.ds(c * P + b * PPS, PPS)],
                        feat_v.at[pl.ds(0, PPS)])

        def gat_body(i, inner):
            w = w_v[pl.ds(i * 16, 16)]
            out_v[pl.ds(i * 16, 16)] = plsc.load_gather(feat_v, [w])
            return inner
        lax.fori_loop(0, RNG // 16, gat_body, 0)
        pltpu.sync_copy(out_v, out.at[pl.ds((b * C + c) * S + lo, RNG)])
        return carry
    lax.fori_loop(0, C, chan_body, 0)


def kernel(pillar_features, voxel_coords):
    feat_t = pillar_features.T.reshape(-1)
    coords = voxel_coords.astype(jnp.int32).reshape(-1)
    out = _pp_scatter(feat_t, coords)
    return out.reshape(BATCH, C * NZ, NY, NX)
